# baseline (speedup 1.0000x reference)
import functools

import jax
import jax.numpy as jnp
from jax import lax
from jax.experimental import pallas as pl
from jax.experimental.pallas import tpu as pltpu
from jax.experimental.pallas import tpu_sc as plsc

B, D, S = 4, 1024, 8192
NW = 32
N_DG = 8
D_PER_W = D // N_DG
N_SR = NW // N_DG
S_PER_W = S // N_SR
S_CHUNK = 128
N_CHUNKS = S_PER_W // S_CHUNK
BP = 2
N_UNITS = N_CHUNKS * (B // BP)
L = 16


def _sc_body(x_hbm, pe_hbm, out_hbm, xt, pet, xsem, psem, osem):
    wid = lax.axis_index("s") * 2 + lax.axis_index("c")
    d0 = (wid % N_DG) * D_PER_W
    s_base = (wid // N_DG) * S_PER_W
    iota = lax.iota(jnp.int32, L)

    def pe_copy(c):
        s0 = s_base + c * S_CHUNK
        return pltpu.make_async_copy(
            pe_hbm.at[pl.ds(s0, S_CHUNK), pl.ds(d0, D_PER_W)],
            pet.at[c % 2], psem.at[c % 2])

    def x_copies(u, p):
        c, h = u // (B // BP), u % (B // BP)
        s0 = s_base + c * S_CHUNK
        return [pltpu.make_async_copy(
            x_hbm.at[h * BP + j, pl.ds(d0, D_PER_W), pl.ds(s0, S_CHUNK)],
            xt.at[p, j], xsem.at[p]) for j in range(BP)]

    def out_copies(u, p):
        c, h = u // (B // BP), u % (B // BP)
        s0 = s_base + c * S_CHUNK
        return [pltpu.make_async_copy(
            xt.at[p, j],
            out_hbm.at[h * BP + j, pl.ds(d0, D_PER_W), pl.ds(s0, S_CHUNK)],
            osem.at[p]) for j in range(BP)]

    pe_copy(0).start()
    for cp in x_copies(0, 0):
        cp.start()

    def unit_body(u, carry):
        p = u % 2
        q = 1 - p
        c = u // (B // BP)

        @pl.when(u >= 1)
        def _():
            for cp in out_copies(u - 1, q):
                cp.wait()

        @pl.when(u + 1 < N_UNITS)
        def _():
            @pl.when((u + 1) % (B // BP) == 0)
            def _():
                pe_copy(c + 1).start()
            for cp in x_copies(u + 1, q):
                cp.start()

        @pl.when(u % (B // BP) == 0)
        def _():
            pe_copy(c).wait()
        for cp in x_copies(u, p):
            cp.wait()

        def d_body(d, carry2):
            d_idx = jnp.zeros((L,), jnp.int32) + d
            for sj in range(S_CHUNK // L):
                s_idx = sj * L + iota
                pv = plsc.load_gather(pet.at[c % 2], [s_idx, d_idx])
                for j in range(BP):
                    plsc.addupdate(xt.at[p, j, d, pl.ds(sj * L, L)], pv)
            return carry2

        lax.fori_loop(0, D_PER_W, d_body, 0)

        for cp in out_copies(u, p):
            cp.start()
        return carry

    lax.fori_loop(0, N_UNITS, unit_body, 0)

    for cp in out_copies(N_UNITS - 1, (N_UNITS - 1) % 2):
        cp.wait()


def kernel(x, pe_table):
    mesh = plsc.VectorSubcoreMesh(core_axis_name="c", subcore_axis_name="s")
    k = functools.partial(
        pl.kernel,
        mesh=mesh,
        out_type=jax.ShapeDtypeStruct((B, D, S), jnp.float32),
        scratch_types=[
            pltpu.VMEM((2, BP, D_PER_W, S_CHUNK), jnp.float32),
            pltpu.VMEM((2, S_CHUNK, D_PER_W), jnp.float32),
            pltpu.SemaphoreType.DMA((2,)),
            pltpu.SemaphoreType.DMA((2,)),
            pltpu.SemaphoreType.DMA((2,)),
        ],
        compiler_params=pltpu.CompilerParams(needs_layout_passes=False),
    )(_sc_body)
    return k(x, pe_table)


# speedup vs baseline: 1.9216x; 1.9216x over previous
import functools

import jax
import jax.numpy as jnp
from jax import lax
from jax.experimental import pallas as pl
from jax.experimental.pallas import tpu as pltpu
from jax.experimental.pallas import tpu_sc as plsc

B, D, S = 4, 1024, 8192
NW = 32
N_DG = 16
D_PER_W = D // N_DG
PE_DW = 128
N_SR = NW // N_DG
S_PER_W = S // N_SR
S_CHUNK = 128
N_CHUNKS = S_PER_W // S_CHUNK
L = 16


def _sc_body(x_hbm, pe_hbm, out_hbm, xt, pet, xsem, psem, osem):
    wid = lax.axis_index("s") * 2 + lax.axis_index("c")
    dg = wid % N_DG
    d0 = dg * D_PER_W
    pe_d0 = (dg // 2) * PE_DW
    d_half = (dg % 2) * D_PER_W
    s_base = (wid // N_DG) * S_PER_W
    iota = lax.iota(jnp.int32, L)

    def pe_copy(c):
        s0 = s_base + c * S_CHUNK
        return pltpu.make_async_copy(
            pe_hbm.at[pl.ds(s0, S_CHUNK), pl.ds(pe_d0, PE_DW)],
            pet.at[c % 2], psem.at[c % 2])

    def x_copies(c, p):
        s0 = s_base + c * S_CHUNK
        return [pltpu.make_async_copy(
            x_hbm.at[b, pl.ds(d0, D_PER_W), pl.ds(s0, S_CHUNK)],
            xt.at[p, b], xsem.at[p]) for b in range(B)]

    def out_copies(c, p):
        s0 = s_base + c * S_CHUNK
        return [pltpu.make_async_copy(
            xt.at[p, b],
            out_hbm.at[b, pl.ds(d0, D_PER_W), pl.ds(s0, S_CHUNK)],
            osem.at[p]) for b in range(B)]

    pe_copy(0).start()
    for cp in x_copies(0, 0):
        cp.start()

    def chunk_body(c, carry):
        p = c % 2
        q = 1 - p

        @pl.when(c >= 1)
        def _():
            for cp in out_copies(c - 1, q):
                cp.wait()

        @pl.when(c + 1 < N_CHUNKS)
        def _():
            pe_copy(c + 1).start()
            for cp in x_copies(c + 1, q):
                cp.start()

        pe_copy(c).wait()
        for cp in x_copies(c, p):
            cp.wait()

        def d_body(d, carry2):
            d_idx = jnp.zeros((L,), jnp.int32) + (d_half + d)
            pvs = [plsc.load_gather(pet.at[p], [sj * L + iota, d_idx])
                   for sj in range(S_CHUNK // L)]
            for b in range(B):
                for sj in range(S_CHUNK // L):
                    plsc.addupdate(xt.at[p, b, d, pl.ds(sj * L, L)], pvs[sj])
            return carry2

        lax.fori_loop(0, D_PER_W, d_body, 0)

        for cp in out_copies(c, p):
            cp.start()
        return carry

    lax.fori_loop(0, N_CHUNKS, chunk_body, 0)

    for cp in out_copies(N_CHUNKS - 1, (N_CHUNKS - 1) % 2):
        cp.wait()


def kernel(x, pe_table):
    mesh = plsc.VectorSubcoreMesh(core_axis_name="c", subcore_axis_name="s")
    k = functools.partial(
        pl.kernel,
        mesh=mesh,
        out_type=jax.ShapeDtypeStruct((B, D, S), jnp.float32),
        scratch_types=[
            pltpu.VMEM((2, B, D_PER_W, S_CHUNK), jnp.float32),
            pltpu.VMEM((2, S_CHUNK, PE_DW), jnp.float32),
            pltpu.SemaphoreType.DMA((2,)),
            pltpu.SemaphoreType.DMA((2,)),
            pltpu.SemaphoreType.DMA((2,)),
        ],
        compiler_params=pltpu.CompilerParams(needs_layout_passes=False),
    )(_sc_body)
    return k(x, pe_table)
